# baseline (speedup 1.0000x reference)
import functools

import jax
import jax.numpy as jnp
from jax import lax
from jax.experimental import pallas as pl
from jax.experimental.pallas import tpu as pltpu
from jax.experimental.pallas import tpu_sc as plsc

_N = 20000
_MAX_OUT = 100
_IOU_T = 0.5
_L = 16
_NTILES = 16
_CHUNK = 1280
_NPAD = _NTILES * _CHUNK
_NSLICE = _CHUNK // _L
_KSLICE = 128 // _L
_NEG = float("-inf")
_BIGI = 2**30


def _fullf(v):
    return jnp.full((_L,), v, jnp.float32)


def _fulli(v):
    return jnp.full((_L,), v, jnp.int32)


def _sc_body(x1_hbm, y1_hbm, x2_hbm, y2_hbm, sc_hbm,
             o1, o2, o3, o4, o5,
             x1, y1, x2, y2, ms,
             kx1, ky1, kx2, ky2, ks, ka,
             rec, table, shared, pad_ref):
    tid = lax.axis_index("s")
    cid = lax.axis_index("c")
    base = tid * _CHUNK
    lanes = lax.iota(jnp.int32, _L)

    pltpu.sync_copy(x1_hbm.at[pl.ds(base, _CHUNK)], x1)
    pltpu.sync_copy(y1_hbm.at[pl.ds(base, _CHUNK)], y1)
    pltpu.sync_copy(x2_hbm.at[pl.ds(base, _CHUNK)], x2)
    pltpu.sync_copy(y2_hbm.at[pl.ds(base, _CHUNK)], y2)
    pltpu.sync_copy(sc_hbm.at[pl.ds(base, _CHUNK)], ms)
    zv = _fullf(0.0)
    for j in range(_KSLICE):
        kx1[pl.ds(j * _L, _L)] = zv
        ky1[pl.ds(j * _L, _L)] = zv
        kx2[pl.ds(j * _L, _L)] = zv
        ky2[pl.ds(j * _L, _L)] = zv
        ks[pl.ds(j * _L, _L)] = zv
        ka[pl.ds(j * _L, _L)] = zv

    def pick1(vec, lane):
        return jnp.sum(jnp.where(lanes == lane, vec, 0.0))

    def body(carry):
        nk, alive, lwi = carry
        lvm = jnp.clip(lwi - base, 0, _CHUNK - 1)
        mret = jnp.logical_and(
            lanes == 0,
            jnp.logical_and(lwi >= 0, lwi // _CHUNK == tid))
        plsc.store_scatter(ms, [_fulli(lvm)], _fullf(_NEG), mask=mret)
        bm = _fullf(_NEG)
        bi = _fulli(0)
        for k in range(_NSLICE):
            v = ms[pl.ds(k * _L, _L)]
            gt = jnp.logical_and(v > bm, lanes != lwi - (base + k * _L))
            bm = jnp.where(gt, v, bm)
            bi = jnp.where(gt, lanes + (base + k * _L), bi)
        lm = jnp.max(bm)
        li = jnp.min(jnp.where(bm == lm, bi, _BIGI))
        lloc = jnp.clip(li - base, 0, _CHUNK - 1)
        gi = _fulli(lloc)
        cx1v = plsc.load_gather(x1, [gi])
        cy1v = plsc.load_gather(y1, [gi])
        cx2v = plsc.load_gather(x2, [gi])
        cy2v = plsc.load_gather(y2, [gi])
        rvec = jnp.where(lanes == 0, _fullf(lm),
               jnp.where(lanes == 1,
                         plsc.bitcast(_fulli(li), jnp.float32),
               jnp.where(lanes == 2, cx1v,
               jnp.where(lanes == 3, cy1v,
               jnp.where(lanes == 4, cx2v,
               jnp.where(lanes == 5, cy2v, _fullf(0.0)))))))
        rec[...] = rvec
        pltpu.sync_copy(rec, shared.at[tid])
        plsc.subcore_barrier()
        pltpu.sync_copy(shared, table)
        plsc.subcore_barrier()
        mvec = plsc.load_gather(table, [lanes, _fulli(0)])
        ivec = plsc.bitcast(plsc.load_gather(table, [lanes, _fulli(1)]),
                            jnp.int32)
        gm = jnp.max(mvec)
        alive = gm > _NEG
        widx = jnp.min(jnp.where(mvec == gm, ivec, _BIGI))
        wrow = _fulli(jnp.clip(widx // _CHUNK, 0, _NTILES - 1))
        wsv = plsc.load_gather(table, [wrow, _fulli(0)])
        wx1 = plsc.load_gather(table, [wrow, _fulli(2)])
        wy1 = plsc.load_gather(table, [wrow, _fulli(3)])
        wx2 = plsc.load_gather(table, [wrow, _fulli(4)])
        wy2 = plsc.load_gather(table, [wrow, _fulli(5)])
        warea = jnp.maximum(wx2 - wx1, 0.0) * jnp.maximum(wy2 - wy1, 0.0)
        mxiou = _fullf(0.0)
        for j in range(_KSLICE):
            s = pl.ds(j * _L, _L)
            xx1 = jnp.maximum(kx1[s], wx1)
            yy1 = jnp.maximum(ky1[s], wy1)
            xx2 = jnp.minimum(kx2[s], wx2)
            yy2 = jnp.minimum(ky2[s], wy2)
            inter = jnp.maximum(xx2 - xx1, 0.0) * jnp.maximum(yy2 - yy1, 0.0)
            iou = inter / (ka[s] + warea - inter + 1e-9)
            mxiou = jnp.maximum(mxiou, iou)
        keep = jnp.logical_and(alive, jnp.max(mxiou) <= _IOU_T)

        kf = jnp.where(keep, _fullf(1.0), _fullf(0.0))
        for j in range(_KSLICE):
            s2 = pl.ds(j * _L, _L)
            lpred = (lanes + j * _L) == nk
            pad_ref[s2] = jnp.where(lpred, kf, pad_ref[s2])

        for j in range(_KSLICE):
            s = pl.ds(j * _L, _L)
            apred = jnp.logical_and(lanes + j * _L == nk, keep)
            kx1[s] = jnp.where(apred, wx1, kx1[s])
            ky1[s] = jnp.where(apred, wy1, ky1[s])
            kx2[s] = jnp.where(apred, wx2, kx2[s])
            ky2[s] = jnp.where(apred, wy2, ky2[s])
            ks[s] = jnp.where(apred, wsv, ks[s])
            ka[s] = jnp.where(apred, warea, ka[s])

        return (nk + keep.astype(jnp.int32), alive, widx)

    lax.while_loop(
        lambda c: jnp.logical_and(c[0] < _MAX_OUT, c[1]),
        body, (jnp.int32(0), jnp.bool_(True), jnp.int32(-1)))

    @pl.when(jnp.logical_and(cid == 0, tid == 0))
    def _():
        pltpu.sync_copy(kx1, o1)
        pltpu.sync_copy(ky1, o2)
        pltpu.sync_copy(kx2, o3)
        pltpu.sync_copy(ky2, o4)
        pltpu.sync_copy(ks, o5)


@jax.jit
def kernel(boxes, scores):
    pad = _NPAD - _N
    x1 = jnp.pad(boxes[:, 0], (0, pad))
    y1 = jnp.pad(boxes[:, 1], (0, pad))
    x2 = jnp.pad(boxes[:, 2], (0, pad))
    y2 = jnp.pad(boxes[:, 3], (0, pad))
    sc = jnp.pad(scores, (0, pad), constant_values=_NEG)
    mesh = plsc.VectorSubcoreMesh(core_axis_name="c", subcore_axis_name="s")
    f128 = jax.ShapeDtypeStruct((128,), jnp.float32)
    run = pl.kernel(
        _sc_body, mesh=mesh,
        compiler_params=pltpu.CompilerParams(needs_layout_passes=False),
        out_type=(f128, f128, f128, f128, f128),
        scratch_types=[
            pltpu.VMEM((_CHUNK,), jnp.float32),
            pltpu.VMEM((_CHUNK,), jnp.float32),
            pltpu.VMEM((_CHUNK,), jnp.float32),
            pltpu.VMEM((_CHUNK,), jnp.float32),
            pltpu.VMEM((_CHUNK,), jnp.float32),
            pltpu.VMEM((128,), jnp.float32),
            pltpu.VMEM((128,), jnp.float32),
            pltpu.VMEM((128,), jnp.float32),
            pltpu.VMEM((128,), jnp.float32),
            pltpu.VMEM((128,), jnp.float32),
            pltpu.VMEM((128,), jnp.float32),
            pltpu.VMEM((_L,), jnp.float32),
            pltpu.VMEM((_NTILES, _L), jnp.float32),
            pltpu.VMEM_SHARED((_NTILES, _L), jnp.float32),
            pltpu.VMEM((128,), jnp.float32),
        ],
    )
    r1, r2, r3, r4, r5 = run(x1, y1, x2, y2, sc)
    res = jnp.stack([r1, r2, r3, r4, r5], axis=1)
    return res[:_MAX_OUT, :]


# speedup vs baseline: 1.1885x; 1.1885x over previous
import functools

import jax
import jax.numpy as jnp
from jax import lax
from jax.experimental import pallas as pl
from jax.experimental.pallas import tpu as pltpu
from jax.experimental.pallas import tpu_sc as plsc

_N = 20000
_MAX_OUT = 100
_IOU_T = 0.5
_L = 16
_NTILES = 16
_CHUNK = 1280
_NPAD = _NTILES * _CHUNK
_NSLICE = _CHUNK // _L
_KSLICE = 128 // _L
_NEG = float("-inf")
_BIGI = 2**30


def _fullf(v):
    return jnp.full((_L,), v, jnp.float32)


def _fulli(v):
    return jnp.full((_L,), v, jnp.int32)


def _sc_body(x1_hbm, y1_hbm, x2_hbm, y2_hbm, sc_hbm,
             o1, o2, o3, o4, o5,
             x1, y1, x2, y2, ms,
             kx1, ky1, kx2, ky2, ks, ka,
             rec, table, shared, pad_ref):
    tid = lax.axis_index("s")
    cid = lax.axis_index("c")
    base = tid * _CHUNK
    lanes = lax.iota(jnp.int32, _L)

    pltpu.sync_copy(x1_hbm.at[pl.ds(base, _CHUNK)], x1)
    pltpu.sync_copy(y1_hbm.at[pl.ds(base, _CHUNK)], y1)
    pltpu.sync_copy(x2_hbm.at[pl.ds(base, _CHUNK)], x2)
    pltpu.sync_copy(y2_hbm.at[pl.ds(base, _CHUNK)], y2)
    pltpu.sync_copy(sc_hbm.at[pl.ds(base, _CHUNK)], ms)
    zv = _fullf(0.0)
    for j in range(_KSLICE):
        kx1[pl.ds(j * _L, _L)] = zv
        ky1[pl.ds(j * _L, _L)] = zv
        kx2[pl.ds(j * _L, _L)] = zv
        ky2[pl.ds(j * _L, _L)] = zv
        ks[pl.ds(j * _L, _L)] = zv
        ka[pl.ds(j * _L, _L)] = zv

    def pick1(vec, lane):
        return jnp.sum(jnp.where(lanes == lane, vec, 0.0))

    def body(carry):
        nk, alive, lgm, lwi = carry
        bm = _fullf(_NEG)
        bi = _fulli(0)
        for k in range(_NSLICE):
            v = ms[pl.ds(k * _L, _L)]
            idxv = lanes + (base + k * _L)
            elig = jnp.logical_or(
                v < lgm,
                jnp.logical_and(v == lgm, idxv > lwi))
            gt = jnp.logical_and(elig, v > bm)
            bm = jnp.where(gt, v, bm)
            bi = jnp.where(gt, idxv, bi)
        lm = jnp.max(bm)
        li = jnp.min(jnp.where(bm == lm, bi, _BIGI))
        lloc = jnp.clip(li - base, 0, _CHUNK - 1)
        gi = _fulli(lloc)
        cx1v = plsc.load_gather(x1, [gi])
        cy1v = plsc.load_gather(y1, [gi])
        cx2v = plsc.load_gather(x2, [gi])
        cy2v = plsc.load_gather(y2, [gi])
        rvec = jnp.where(lanes == 0, _fullf(lm),
               jnp.where(lanes == 1,
                         plsc.bitcast(_fulli(li), jnp.float32),
               jnp.where(lanes == 2, cx1v,
               jnp.where(lanes == 3, cy1v,
               jnp.where(lanes == 4, cx2v,
               jnp.where(lanes == 5, cy2v, _fullf(0.0)))))))
        rec[...] = rvec
        pltpu.sync_copy(rec, shared.at[tid])
        plsc.subcore_barrier()
        pltpu.sync_copy(shared, table)
        plsc.subcore_barrier()
        mvec = plsc.load_gather(table, [lanes, _fulli(0)])
        ivec = plsc.bitcast(plsc.load_gather(table, [lanes, _fulli(1)]),
                            jnp.int32)
        gm = jnp.max(mvec)
        alive = gm > _NEG
        widx = jnp.min(jnp.where(mvec == gm, ivec, _BIGI))
        wrow = _fulli(jnp.clip(widx // _CHUNK, 0, _NTILES - 1))
        wsv = plsc.load_gather(table, [wrow, _fulli(0)])
        wx1 = plsc.load_gather(table, [wrow, _fulli(2)])
        wy1 = plsc.load_gather(table, [wrow, _fulli(3)])
        wx2 = plsc.load_gather(table, [wrow, _fulli(4)])
        wy2 = plsc.load_gather(table, [wrow, _fulli(5)])
        warea = jnp.maximum(wx2 - wx1, 0.0) * jnp.maximum(wy2 - wy1, 0.0)
        mxiou = _fullf(0.0)
        for j in range(_KSLICE - 1):
            s = pl.ds(j * _L, _L)
            xx1 = jnp.maximum(kx1[s], wx1)
            yy1 = jnp.maximum(ky1[s], wy1)
            xx2 = jnp.minimum(kx2[s], wx2)
            yy2 = jnp.minimum(ky2[s], wy2)
            inter = jnp.maximum(xx2 - xx1, 0.0) * jnp.maximum(yy2 - yy1, 0.0)
            iou = inter / (ka[s] + warea - inter + 1e-9)
            mxiou = jnp.maximum(mxiou, iou)
        keep = jnp.logical_and(alive, jnp.max(mxiou) <= _IOU_T)

        kf = jnp.where(keep, _fullf(1.0), _fullf(0.0))
        for j in range(_KSLICE):
            s2 = pl.ds(j * _L, _L)
            lpred = (lanes + j * _L) == nk
            pad_ref[s2] = jnp.where(lpred, kf, pad_ref[s2])

        for j in range(_KSLICE - 1):
            s = pl.ds(j * _L, _L)
            apred = jnp.logical_and(lanes + j * _L == nk, keep)
            kx1[s] = jnp.where(apred, wx1, kx1[s])
            ky1[s] = jnp.where(apred, wy1, ky1[s])
            kx2[s] = jnp.where(apred, wx2, kx2[s])
            ky2[s] = jnp.where(apred, wy2, ky2[s])
            ks[s] = jnp.where(apred, wsv, ks[s])
            ka[s] = jnp.where(apred, warea, ka[s])

        return (nk + keep.astype(jnp.int32), alive, gm, widx)

    lax.while_loop(
        lambda c: jnp.logical_and(c[0] < _MAX_OUT, c[1]),
        body, (jnp.int32(0), jnp.bool_(True), jnp.float32(jnp.inf),
               jnp.int32(-1)))

    @pl.when(jnp.logical_and(cid == 0, tid == 0))
    def _():
        pltpu.sync_copy(kx1, o1)
        pltpu.sync_copy(ky1, o2)
        pltpu.sync_copy(kx2, o3)
        pltpu.sync_copy(ky2, o4)
        pltpu.sync_copy(ks, o5)


@jax.jit
def kernel(boxes, scores):
    pad = _NPAD - _N
    x1 = jnp.pad(boxes[:, 0], (0, pad))
    y1 = jnp.pad(boxes[:, 1], (0, pad))
    x2 = jnp.pad(boxes[:, 2], (0, pad))
    y2 = jnp.pad(boxes[:, 3], (0, pad))
    sc = jnp.pad(scores, (0, pad), constant_values=_NEG)
    mesh = plsc.VectorSubcoreMesh(core_axis_name="c", subcore_axis_name="s")
    f128 = jax.ShapeDtypeStruct((128,), jnp.float32)
    run = pl.kernel(
        _sc_body, mesh=mesh,
        compiler_params=pltpu.CompilerParams(needs_layout_passes=False),
        out_type=(f128, f128, f128, f128, f128),
        scratch_types=[
            pltpu.VMEM((_CHUNK,), jnp.float32),
            pltpu.VMEM((_CHUNK,), jnp.float32),
            pltpu.VMEM((_CHUNK,), jnp.float32),
            pltpu.VMEM((_CHUNK,), jnp.float32),
            pltpu.VMEM((_CHUNK,), jnp.float32),
            pltpu.VMEM((128,), jnp.float32),
            pltpu.VMEM((128,), jnp.float32),
            pltpu.VMEM((128,), jnp.float32),
            pltpu.VMEM((128,), jnp.float32),
            pltpu.VMEM((128,), jnp.float32),
            pltpu.VMEM((128,), jnp.float32),
            pltpu.VMEM((_L,), jnp.float32),
            pltpu.VMEM((_NTILES, _L), jnp.float32),
            pltpu.VMEM_SHARED((_NTILES, _L), jnp.float32),
            pltpu.VMEM((128,), jnp.float32),
        ],
    )
    r1, r2, r3, r4, r5 = run(x1, y1, x2, y2, sc)
    res = jnp.stack([r1, r2, r3, r4, r5], axis=1)
    return res[:_MAX_OUT, :]


# speedup vs baseline: 1.1913x; 1.0024x over previous
import functools

import jax
import jax.numpy as jnp
from jax import lax
from jax.experimental import pallas as pl
from jax.experimental.pallas import tpu as pltpu
from jax.experimental.pallas import tpu_sc as plsc

_N = 20000
_MAX_OUT = 100
_IOU_T = 0.5
_L = 16
_NTILES = 16
_CHUNK = 1280
_NPAD = _NTILES * _CHUNK
_NSLICE = _CHUNK // _L
_KSLICE = 128 // _L
_NEG = float("-inf")
_BIGI = 2**30


def _fullf(v):
    return jnp.full((_L,), v, jnp.float32)


def _fulli(v):
    return jnp.full((_L,), v, jnp.int32)


def _sc_body(x1_hbm, y1_hbm, x2_hbm, y2_hbm, sc_hbm,
             o1, o2, o3, o4, o5,
             x1, y1, x2, y2, ms,
             kx1, ky1, kx2, ky2, ks, ka,
             rec, table, shared, pad_ref):
    tid = lax.axis_index("s")
    cid = lax.axis_index("c")
    base = tid * _CHUNK
    lanes = lax.iota(jnp.int32, _L)

    pltpu.sync_copy(x1_hbm.at[pl.ds(base, _CHUNK)], x1)
    pltpu.sync_copy(y1_hbm.at[pl.ds(base, _CHUNK)], y1)
    pltpu.sync_copy(x2_hbm.at[pl.ds(base, _CHUNK)], x2)
    pltpu.sync_copy(y2_hbm.at[pl.ds(base, _CHUNK)], y2)
    pltpu.sync_copy(sc_hbm.at[pl.ds(base, _CHUNK)], ms)
    zv = _fullf(0.0)
    for j in range(_KSLICE):
        kx1[pl.ds(j * _L, _L)] = zv
        ky1[pl.ds(j * _L, _L)] = zv
        kx2[pl.ds(j * _L, _L)] = zv
        ky2[pl.ds(j * _L, _L)] = zv
        ks[pl.ds(j * _L, _L)] = zv
        ka[pl.ds(j * _L, _L)] = zv

    def pick1(vec, lane):
        return jnp.sum(jnp.where(lanes == lane, vec, 0.0))

    def body(carry):
        nk, alive, lgm, lwi = carry
        bm = _fullf(_NEG)
        bi = _fulli(0)
        for k in range(_NSLICE):
            v = ms[pl.ds(k * _L, _L)]
            idxv = lanes + (base + k * _L)
            elig = jnp.logical_or(
                v < lgm,
                jnp.logical_and(v == lgm, idxv > lwi))
            gt = jnp.logical_and(elig, v > bm)
            bm = jnp.where(gt, v, bm)
            bi = jnp.where(gt, idxv, bi)
        lm = jnp.max(bm)
        li = jnp.min(jnp.where(bm == lm, bi, _BIGI))
        lloc = jnp.clip(li - base, 0, _CHUNK - 1)
        gi = _fulli(lloc)
        cx1v = plsc.load_gather(x1, [gi])
        cy1v = plsc.load_gather(y1, [gi])
        cx2v = plsc.load_gather(x2, [gi])
        cy2v = plsc.load_gather(y2, [gi])
        rvec = jnp.where(lanes == 0, _fullf(lm),
               jnp.where(lanes == 1,
                         plsc.bitcast(_fulli(li), jnp.float32),
               jnp.where(lanes == 2, cx1v,
               jnp.where(lanes == 3, cy1v,
               jnp.where(lanes == 4, cx2v,
               jnp.where(lanes == 5, cy2v, _fullf(0.0)))))))
        rec[...] = rvec
        pltpu.sync_copy(rec, shared.at[tid])
        plsc.subcore_barrier()
        pltpu.sync_copy(shared, table)
        plsc.subcore_barrier()
        mvec = plsc.load_gather(table, [lanes, _fulli(0)])
        ivec = plsc.bitcast(plsc.load_gather(table, [lanes, _fulli(1)]),
                            jnp.int32)
        gm = jnp.max(mvec)
        alive = gm > _NEG
        widx = jnp.min(jnp.where(mvec == gm, ivec, _BIGI))
        wrow = _fulli(jnp.clip(widx // _CHUNK, 0, _NTILES - 1))
        wsv = plsc.load_gather(table, [wrow, _fulli(0)])
        wx1 = plsc.load_gather(table, [wrow, _fulli(2)])
        wy1 = plsc.load_gather(table, [wrow, _fulli(3)])
        wx2 = plsc.load_gather(table, [wrow, _fulli(4)])
        wy2 = plsc.load_gather(table, [wrow, _fulli(5)])
        warea = jnp.maximum(wx2 - wx1, 0.0) * jnp.maximum(wy2 - wy1, 0.0)
        mxiou = _fullf(0.0)
        for j in range(_KSLICE - 1):
            s = pl.ds(j * _L, _L)
            xx1 = jnp.maximum(kx1[s], wx1)
            yy1 = jnp.maximum(ky1[s], wy1)
            xx2 = jnp.minimum(kx2[s], wx2)
            yy2 = jnp.minimum(ky2[s], wy2)
            inter = jnp.maximum(xx2 - xx1, 0.0) * jnp.maximum(yy2 - yy1, 0.0)
            iou = inter / (ka[s] + warea - inter + 1e-9)
            mxiou = jnp.maximum(mxiou, iou)
        keep = jnp.logical_and(alive, jnp.max(mxiou) <= _IOU_T)

        kf = jnp.where(keep, _fullf(1.0), _fullf(0.0))
        for j in range(4):
            s2 = pl.ds(j * _L, _L)
            lpred = (lanes + j * _L) == nk
            pad_ref[s2] = jnp.where(lpred, kf, pad_ref[s2])

        for j in range(_KSLICE - 1):
            s = pl.ds(j * _L, _L)
            apred = jnp.logical_and(lanes + j * _L == nk, keep)
            kx1[s] = jnp.where(apred, wx1, kx1[s])
            ky1[s] = jnp.where(apred, wy1, ky1[s])
            kx2[s] = jnp.where(apred, wx2, kx2[s])
            ky2[s] = jnp.where(apred, wy2, ky2[s])
            ks[s] = jnp.where(apred, wsv, ks[s])
            ka[s] = jnp.where(apred, warea, ka[s])

        return (nk + keep.astype(jnp.int32), alive, gm, widx)

    lax.while_loop(
        lambda c: jnp.logical_and(c[0] < _MAX_OUT, c[1]),
        body, (jnp.int32(0), jnp.bool_(True), jnp.float32(jnp.inf),
               jnp.int32(-1)))

    @pl.when(jnp.logical_and(cid == 0, tid == 0))
    def _():
        pltpu.sync_copy(kx1, o1)
        pltpu.sync_copy(ky1, o2)
        pltpu.sync_copy(kx2, o3)
        pltpu.sync_copy(ky2, o4)
        pltpu.sync_copy(ks, o5)


@jax.jit
def kernel(boxes, scores):
    pad = _NPAD - _N
    x1 = jnp.pad(boxes[:, 0], (0, pad))
    y1 = jnp.pad(boxes[:, 1], (0, pad))
    x2 = jnp.pad(boxes[:, 2], (0, pad))
    y2 = jnp.pad(boxes[:, 3], (0, pad))
    sc = jnp.pad(scores, (0, pad), constant_values=_NEG)
    mesh = plsc.VectorSubcoreMesh(core_axis_name="c", subcore_axis_name="s")
    f128 = jax.ShapeDtypeStruct((128,), jnp.float32)
    run = pl.kernel(
        _sc_body, mesh=mesh,
        compiler_params=pltpu.CompilerParams(needs_layout_passes=False),
        out_type=(f128, f128, f128, f128, f128),
        scratch_types=[
            pltpu.VMEM((_CHUNK,), jnp.float32),
            pltpu.VMEM((_CHUNK,), jnp.float32),
            pltpu.VMEM((_CHUNK,), jnp.float32),
            pltpu.VMEM((_CHUNK,), jnp.float32),
            pltpu.VMEM((_CHUNK,), jnp.float32),
            pltpu.VMEM((128,), jnp.float32),
            pltpu.VMEM((128,), jnp.float32),
            pltpu.VMEM((128,), jnp.float32),
            pltpu.VMEM((128,), jnp.float32),
            pltpu.VMEM((128,), jnp.float32),
            pltpu.VMEM((128,), jnp.float32),
            pltpu.VMEM((_L,), jnp.float32),
            pltpu.VMEM((_NTILES, _L), jnp.float32),
            pltpu.VMEM_SHARED((_NTILES, _L), jnp.float32),
            pltpu.VMEM((128,), jnp.float32),
        ],
    )
    r1, r2, r3, r4, r5 = run(x1, y1, x2, y2, sc)
    res = jnp.stack([r1, r2, r3, r4, r5], axis=1)
    return res[:_MAX_OUT, :]
